# baseline (speedup 1.0000x reference)
import functools

import jax
import jax.numpy as jnp
from jax.experimental import pallas as pl
from jax.experimental.pallas import tpu as pltpu

_BB = 512
_BR = 1024


def _split_hi_lo(v):
    hi = v.astype(jnp.bfloat16)
    lo = (v - hi.astype(jnp.float32)).astype(jnp.bfloat16)
    return hi, lo


def _mwc_kernel(
    nb, nr,
    eo_ref, ec_ref, c_ref, idx_ref, eps_ref, out_ref,
    ph_scr, plo_scr, dh_scr, dlo_scr, s_scr, er_scr, x_scr, sd_scr,
):
    i = pl.program_id(0)
    n_steps = nb * nr
    n_units = eo_ref.shape[1]
    br = _BR
    ii = jnp.minimum(i, n_steps - 1)
    ir = ii % nr
    par = (i % 2) * _BB

    def build_slice(sl):
        idx = idx_ref[:, pl.ds(sl * br, br)]
        u_iota = jax.lax.broadcasted_iota(jnp.int32, (n_units, br), 0)
        s = jnp.zeros((n_units, br), jnp.float32)
        for k in range(idx_ref.shape[0]):
            s = s + jnp.where(u_iota == idx[k : k + 1, :], 1.0, 0.0)
        sb = s.astype(jnp.bfloat16)
        s_scr[:, pl.ds(sl * br, br)] = sb
        eh, elo = _split_hi_lo(eps_ref[...])
        er = jnp.dot(eh, sb, preferred_element_type=jnp.float32) + jnp.dot(
            elo, sb, preferred_element_type=jnp.float32
        )
        er_scr[0:1, pl.ds(sl * br, br)] = er

    @pl.when(jnp.logical_and(i < n_steps, ir == 0))
    def _():
        c = c_ref[...]
        eo = eo_ref[...]
        ec = ec_ref[...]
        p = jnp.log1p(c * jnp.exp(-ec)) - jnp.log1p(c * jnp.exp(-eo))
        ph_scr[...], plo_scr[...] = _split_hi_lo(p)
        dh_scr[...], dlo_scr[...] = _split_hi_lo(eo - ec)

    @pl.when(i == 0)
    def _():
        build_slice(0)

    def epilogue():
        prev = _BB - par
        irp = (i - 1) % nr
        x = x_scr[pl.ds(prev, _BB), :]
        sd = sd_scr[pl.ds(prev, _BB), :]
        er = er_scr[0:1, pl.ds(irp * br, br)]
        L = jnp.exp(-er)
        p_min = 1.0 / (1.0 + L)
        p_c = 1.0 / (1.0 + L * jnp.exp(x))
        p_max = 1.0 / (1.0 + L * jnp.exp(sd))
        denom = p_max - p_min
        norm = (p_c - p_min) / (denom + 1e-8)
        norm = jnp.where(denom > 1e-6, norm, 0.0)
        out_ref[...] = jnp.clip(norm, 0.0, 1.0)

    @pl.when(i < n_steps)
    def _():
        sb = s_scr[:, pl.ds(ir * br, br)]
        x_scr[pl.ds(par, _BB), :] = jnp.dot(
            ph_scr[...], sb, preferred_element_type=jnp.float32
        ) + jnp.dot(plo_scr[...], sb, preferred_element_type=jnp.float32)
        sd_scr[pl.ds(par, _BB), :] = jnp.dot(
            dh_scr[...], sb, preferred_element_type=jnp.float32
        ) + jnp.dot(dlo_scr[...], sb, preferred_element_type=jnp.float32)
        build_slice((ir + 1) % nr)
        epilogue()

    @pl.when(i == n_steps)
    def _():
        epilogue()


@jax.jit
def kernel(energies, concentrations, receptor_indices, epsilon_units):
    b, u, _ = energies.shape
    r, k = receptor_indices.shape
    bb = _BB
    br = _BR
    nb = b // bb
    nr = r // br
    n_steps = nb * nr

    one = jnp.float32(1.0)
    eo = energies[:, :, 0] * one
    ec = energies[:, :, 1] * one
    c2 = concentrations.reshape(b, 1)
    idxt = receptor_indices.T
    eps2 = epsilon_units.reshape(1, u)

    body = functools.partial(_mwc_kernel, nb, nr)

    def clamp(i):
        return jnp.minimum(i, n_steps - 1)

    def lag(i):
        return jnp.maximum(i - 1, 0)

    return pl.pallas_call(
        body,
        grid=(n_steps + 1,),
        in_specs=[
            pl.BlockSpec((bb, u), lambda i: (clamp(i) // nr, 0)),
            pl.BlockSpec((bb, u), lambda i: (clamp(i) // nr, 0)),
            pl.BlockSpec((bb, 1), lambda i: (clamp(i) // nr, 0)),
            pl.BlockSpec((k, r), lambda i: (0, 0)),
            pl.BlockSpec((1, u), lambda i: (0, 0)),
        ],
        out_specs=pl.BlockSpec((bb, br), lambda i: (lag(i) // nr, lag(i) % nr)),
        out_shape=jax.ShapeDtypeStruct((b, r), jnp.float32),
        scratch_shapes=[
            pltpu.VMEM((bb, u), jnp.bfloat16),
            pltpu.VMEM((bb, u), jnp.bfloat16),
            pltpu.VMEM((bb, u), jnp.bfloat16),
            pltpu.VMEM((bb, u), jnp.bfloat16),
            pltpu.VMEM((u, r), jnp.bfloat16),
            pltpu.VMEM((8, r), jnp.float32),
            pltpu.VMEM((2 * bb, br), jnp.float32),
            pltpu.VMEM((2 * bb, br), jnp.float32),
        ],
    )(eo, ec, c2, idxt, eps2)


# speedup vs baseline: 1.2615x; 1.2615x over previous
import jax
import jax.numpy as jnp
from jax.experimental import pallas as pl
from jax.experimental.pallas import tpu as pltpu


def _split_hi_lo(v):
    hi = v.astype(jnp.bfloat16)
    lo = (v - hi.astype(jnp.float32)).astype(jnp.bfloat16)
    return hi, lo


def _mwc_kernel(
    eo_ref, ec_ref, c_ref, idx_ref, eps_ref, out_ref,
    ph_scr, plo_scr, dh_scr, dlo_scr, s_scr,
):
    ib = pl.program_id(0)
    ir = pl.program_id(1)
    n_units = eo_ref.shape[1]
    br = out_ref.shape[1]

    @pl.when(ir == 0)
    def _():
        c = c_ref[...]
        eo = eo_ref[...]
        ec = ec_ref[...]
        p = jnp.log1p(c * jnp.exp(-ec)) - jnp.log1p(c * jnp.exp(-eo))
        ph, plo = _split_hi_lo(p)
        eh, elo = _split_hi_lo(eps_ref[...])
        zpad = jnp.zeros((7, n_units), jnp.bfloat16)
        ph_scr[...] = jnp.concatenate([ph, eh, zpad], axis=0)
        plo_scr[...] = jnp.concatenate([plo, elo, zpad], axis=0)
        dh_scr[...], dlo_scr[...] = _split_hi_lo(eo - ec)

    @pl.when(ib == 0)
    def _():
        idx = idx_ref[...]
        u_iota = jax.lax.broadcasted_iota(jnp.int32, (n_units, br), 0)
        s = jnp.zeros((n_units, br), jnp.float32)
        for k in range(idx_ref.shape[0]):
            s = s + jnp.where(u_iota == idx[k : k + 1, :], 1.0, 0.0)
        s_scr[:, pl.ds(ir * br, br)] = s.astype(jnp.bfloat16)

    sb = s_scr[:, pl.ds(ir * br, br)]
    xf = jnp.dot(ph_scr[...], sb, preferred_element_type=jnp.float32) + jnp.dot(
        plo_scr[...], sb, preferred_element_type=jnp.float32
    )
    x = xf[: out_ref.shape[0], :]
    er = xf[out_ref.shape[0] : out_ref.shape[0] + 1, :]
    sd = jnp.dot(dh_scr[...], sb, preferred_element_type=jnp.float32) + jnp.dot(
        dlo_scr[...], sb, preferred_element_type=jnp.float32
    )

    L = jnp.exp(-er)
    p_min = 1.0 / (1.0 + L)
    p_c = 1.0 / (1.0 + L * jnp.exp(x))
    p_max = 1.0 / (1.0 + L * jnp.exp(sd))
    denom = p_max - p_min
    norm = (p_c - p_min) / (denom + 1e-8)
    norm = jnp.where(denom > 1e-6, norm, 0.0)
    out_ref[...] = jnp.clip(norm, 0.0, 1.0)


@jax.jit
def kernel(energies, concentrations, receptor_indices, epsilon_units):
    b, u, _ = energies.shape
    r, k = receptor_indices.shape
    bb = 512
    br = 1024
    nb = b // bb
    nr = r // br

    one = jnp.float32(1.0)
    eo = energies[:, :, 0] * one
    ec = energies[:, :, 1] * one
    c2 = concentrations.reshape(b, 1)
    idxt = receptor_indices.T
    eps2 = epsilon_units.reshape(1, u)

    return pl.pallas_call(
        _mwc_kernel,
        grid=(nb, nr),
        in_specs=[
            pl.BlockSpec((bb, u), lambda ib, ir: (ib, 0)),
            pl.BlockSpec((bb, u), lambda ib, ir: (ib, 0)),
            pl.BlockSpec((bb, 1), lambda ib, ir: (ib, 0)),
            pl.BlockSpec((k, br), lambda ib, ir: (0, ir)),
            pl.BlockSpec((1, u), lambda ib, ir: (0, 0)),
        ],
        out_specs=pl.BlockSpec((bb, br), lambda ib, ir: (ib, ir)),
        out_shape=jax.ShapeDtypeStruct((b, r), jnp.float32),
        scratch_shapes=[
            pltpu.VMEM((bb + 8, u), jnp.bfloat16),
            pltpu.VMEM((bb + 8, u), jnp.bfloat16),
            pltpu.VMEM((bb, u), jnp.bfloat16),
            pltpu.VMEM((bb, u), jnp.bfloat16),
            pltpu.VMEM((u, r), jnp.bfloat16),
        ],
    )(eo, ec, c2, idxt, eps2)


# speedup vs baseline: 1.2805x; 1.0150x over previous
import jax
import jax.numpy as jnp
from jax.experimental import pallas as pl
from jax.experimental.pallas import tpu as pltpu


def _split_hi_lo(v):
    hi = v.astype(jnp.bfloat16)
    lo = (v - hi.astype(jnp.float32)).astype(jnp.bfloat16)
    return hi, lo


def _mwc_kernel(
    eo_ref, ec_ref, c_ref, idx_ref, eps_ref, out_ref,
    ph_scr, plo_scr, dh_scr, dlo_scr,
):
    ir = pl.program_id(0)
    n_units = eo_ref.shape[1]
    bb = out_ref.shape[0]
    br = out_ref.shape[1]

    @pl.when(ir == 0)
    def _():
        c = c_ref[...]
        eo = eo_ref[...]
        ec = ec_ref[...]
        p = jnp.log1p(c * jnp.exp(-ec)) - jnp.log1p(c * jnp.exp(-eo))
        ph, plo = _split_hi_lo(p)
        eh, elo = _split_hi_lo(eps_ref[...])
        zpad = jnp.zeros((7, n_units), jnp.bfloat16)
        ph_scr[...] = jnp.concatenate([ph, eh, zpad], axis=0)
        plo_scr[...] = jnp.concatenate([plo, elo, zpad], axis=0)
        dh_scr[...], dlo_scr[...] = _split_hi_lo(eo - ec)

    idx = idx_ref[...]
    u_iota = jax.lax.broadcasted_iota(jnp.int32, (n_units, br), 0)
    s = jnp.zeros((n_units, br), jnp.float32)
    for k in range(idx_ref.shape[0]):
        s = s + jnp.where(u_iota == idx[k : k + 1, :], 1.0, 0.0)
    sb = s.astype(jnp.bfloat16)

    xf = jnp.dot(ph_scr[...], sb, preferred_element_type=jnp.float32) + jnp.dot(
        plo_scr[...], sb, preferred_element_type=jnp.float32
    )
    x = xf[:bb, :]
    er = xf[bb : bb + 1, :]
    sd = jnp.dot(dh_scr[...], sb, preferred_element_type=jnp.float32) + jnp.dot(
        dlo_scr[...], sb, preferred_element_type=jnp.float32
    )

    L = jnp.exp(-er)
    p_min = 1.0 / (1.0 + L)
    p_c = 1.0 / (1.0 + L * jnp.exp(x))
    p_max = 1.0 / (1.0 + L * jnp.exp(sd))
    denom = p_max - p_min
    norm = (p_c - p_min) / (denom + 1e-8)
    norm = jnp.where(denom > 1e-6, norm, 0.0)
    out_ref[...] = jnp.clip(norm, 0.0, 1.0)


@jax.jit
def kernel(energies, concentrations, receptor_indices, epsilon_units):
    b, u, _ = energies.shape
    r, k = receptor_indices.shape
    br = 512
    nr = r // br

    one = jnp.float32(1.0)
    eo = energies[:, :, 0] * one
    ec = energies[:, :, 1] * one
    c2 = concentrations.reshape(b, 1)
    idxt = receptor_indices.T
    eps2 = epsilon_units.reshape(1, u)

    return pl.pallas_call(
        _mwc_kernel,
        grid=(nr,),
        in_specs=[
            pl.BlockSpec((b, u), lambda ir: (0, 0)),
            pl.BlockSpec((b, u), lambda ir: (0, 0)),
            pl.BlockSpec((b, 1), lambda ir: (0, 0)),
            pl.BlockSpec((k, br), lambda ir: (0, ir)),
            pl.BlockSpec((1, u), lambda ir: (0, 0)),
        ],
        out_specs=pl.BlockSpec((b, br), lambda ir: (0, ir)),
        out_shape=jax.ShapeDtypeStruct((b, r), jnp.float32),
        scratch_shapes=[
            pltpu.VMEM((b + 8, u), jnp.bfloat16),
            pltpu.VMEM((b + 8, u), jnp.bfloat16),
            pltpu.VMEM((b, u), jnp.bfloat16),
            pltpu.VMEM((b, u), jnp.bfloat16),
        ],
    )(eo, ec, c2, idxt, eps2)
